# baseline (speedup 1.0000x reference)
import functools
import math

import jax
import jax.numpy as jnp
import numpy as np
from jax import lax
from jax.experimental import pallas as pl
from jax.experimental.pallas import tpu as pltpu
from jax.experimental.pallas import tpu_sc as plsc

VOCAB = 1000000
EMBED_DIM = 64
SEQ_PROJ_DIM = 128
BATCH = 1024
SEQ_LEN = 200

NUM_IDX = BATCH * SEQ_LEN

_NC, _NS = 2, 16
_NW = _NC * _NS
_B_PER_W = NUM_IDX // _NW
_CHUNK = 640
_N_CHUNKS = _B_PER_W // _CHUNK

_B_BLK = 16
_ROWS_BLK = _B_BLK * SEQ_LEN


def _positional_encoding(seq_len, embed_dim):
    position = np.arange(0, seq_len, dtype=np.float32)[:, None]
    div_term = np.exp(
        np.arange(0, embed_dim, 2, dtype=np.float32) * (-math.log(10000.0) / embed_dim)
    )
    pe = np.zeros((seq_len, embed_dim), dtype=np.float32)
    pe[:, 0::2] = np.sin(position * div_term)
    pe[:, 1::2] = np.cos(position * div_term)
    return pe


def _sc_gather(table, idx2d):
    mesh = plsc.VectorSubcoreMesh(core_axis_name="c", subcore_axis_name="s")

    @functools.partial(
        pl.kernel,
        mesh=mesh,
        out_type=jax.ShapeDtypeStruct((NUM_IDX, EMBED_DIM), jnp.float32),
        scratch_types=[
            pltpu.VMEM((_CHUNK,), jnp.int32),
            pltpu.VMEM((_CHUNK, EMBED_DIM), jnp.float32),
            pltpu.SemaphoreType.DMA,
        ],
        compiler_params=pltpu.CompilerParams(use_tc_tiling_on_sc=False),
    )
    def k(table_hbm, idx_hbm, out_hbm, idx_v, rows_v, sem):
        wid = lax.axis_index("s") * _NC + lax.axis_index("c")
        base = wid * _B_PER_W

        @pl.loop(0, _N_CHUNKS)
        def _(j):
            pltpu.sync_copy(idx_hbm.at[wid * _N_CHUNKS + j], idx_v)
            pltpu.async_copy(table_hbm.at[idx_v], rows_v, sem).wait()
            pltpu.sync_copy(rows_v, out_hbm.at[pl.ds(base + j * _CHUNK, _CHUNK)])

    return k(table, idx2d)


def _layernorm_proj(e, gm, bt, w, b2):
    mean = jnp.mean(e, axis=1, keepdims=True)
    c = e - mean
    var = jnp.mean(c * c, axis=1, keepdims=True)
    z = c * lax.rsqrt(var + 1e-5)
    z = z * gm + bt
    return jnp.dot(z, w, preferred_element_type=jnp.float32) + b2


def _tc_fuse(gathered2, pe_l, pe_r, gamma, beta, W, b):
    def body(g_ref, pel_ref, per_ref, gm_ref, bt_ref, w_ref, b_ref, o_ref):
        g = g_ref[...]
        gm, bt, w, b2 = gm_ref[...], bt_ref[...], w_ref[...], b_ref[...]
        yl = _layernorm_proj(g[:, :EMBED_DIM] + pel_ref[...], gm, bt, w, b2)
        yr = _layernorm_proj(g[:, EMBED_DIM:] + per_ref[...], gm, bt, w, b2)
        res = jnp.concatenate([yl[:, None, :], yr[:, None, :]], axis=1)
        o_ref[...] = res.reshape(_B_BLK, SEQ_LEN, SEQ_PROJ_DIM)

    half_blk = _ROWS_BLK // 2
    return pl.pallas_call(
        body,
        grid=(NUM_IDX // _ROWS_BLK,),
        in_specs=[
            pl.BlockSpec((half_blk, 2 * EMBED_DIM), lambda i: (i, 0)),
            pl.BlockSpec((half_blk, EMBED_DIM), lambda i: (0, 0)),
            pl.BlockSpec((half_blk, EMBED_DIM), lambda i: (0, 0)),
            pl.BlockSpec((1, EMBED_DIM), lambda i: (0, 0)),
            pl.BlockSpec((1, EMBED_DIM), lambda i: (0, 0)),
            pl.BlockSpec((EMBED_DIM, SEQ_PROJ_DIM), lambda i: (0, 0)),
            pl.BlockSpec((1, SEQ_PROJ_DIM), lambda i: (0, 0)),
        ],
        out_specs=pl.BlockSpec((_B_BLK, SEQ_LEN, SEQ_PROJ_DIM), lambda i: (i, 0, 0)),
        out_shape=jax.ShapeDtypeStruct((BATCH, SEQ_LEN, SEQ_PROJ_DIM), jnp.float32),
    )(gathered2, pe_l, pe_r, gamma.reshape(1, -1), beta.reshape(1, -1), W,
      b.reshape(1, -1))


def kernel(x, table, gamma, beta, W, b):
    idx2d = x.astype(jnp.int32).reshape(_NW * _N_CHUNKS, _CHUNK)
    gathered = _sc_gather(table, idx2d)
    gathered2 = gathered.reshape(NUM_IDX // 2, 2 * EMBED_DIM)
    pe = _positional_encoding(SEQ_LEN, EMBED_DIM)
    half_blk = _ROWS_BLK // 2
    pos = np.arange(half_blk)
    pe_l = jnp.asarray(pe[(2 * pos) % SEQ_LEN])
    pe_r = jnp.asarray(pe[(2 * pos + 1) % SEQ_LEN])
    return _tc_fuse(gathered2, pe_l, pe_r, gamma, beta, W, b)
